# baseline (speedup 1.0000x reference)
import functools

import jax
import jax.numpy as jnp
from jax.experimental import pallas as pl
from jax.experimental.pallas import tpu as pltpu

H = W = 16
C = 64
SP = 289
HIDDEN = 32
NG = 3 * HIDDEN
EPS = 1e-5
TBL = 256


def _fused_kernel(x_ref, w2_ref, wg_ref, bg_ref, sc_ref, wfc_ref, bfc_ref,
                  o_ref):
    xbf = x_ref[...].astype(jnp.bfloat16)
    xb = xbf.astype(jnp.float32)
    row = jax.lax.broadcasted_iota(jnp.int32, (256, TBL), 0)
    sub = row % 16
    zero = jnp.zeros_like(xb)

    s10 = jnp.where(sub != 0, jnp.roll(xb, 1, axis=0), zero)
    s01 = jnp.where(row >= 16, jnp.roll(xb, 16, axis=0), zero)
    s00 = jnp.where((row >= 16) & (sub != 0), jnp.roll(xb, 17, axis=0), zero)
    s1m = jnp.where((row >= 16) & (sub != 15), jnp.roll(xb, 15, axis=0), zero)
    ssum = jnp.sum(xb, axis=0, keepdims=True)
    c00 = jnp.sum(xb * xb, axis=0, keepdims=True)
    c01 = jnp.sum(xb * s10, axis=0, keepdims=True)
    c10 = jnp.sum(xb * s01, axis=0, keepdims=True)
    c11 = jnp.sum(xb * s00, axis=0, keepdims=True)
    c1m = jnp.sum(xb * s1m, axis=0, keepdims=True)

    mean = sc_ref[:, 0:1] * ssum
    ex2 = (sc_ref[:, 1:2] * c00 + sc_ref[:, 2:3] * c01
           + sc_ref[:, 3:4] * c10 + sc_ref[:, 4:5] * c11
           + sc_ref[:, 5:6] * c1m)
    scale = jax.lax.rsqrt(ex2 - mean * mean + EPS)
    mean_t = jnp.tile(mean, (4, 1))
    scale_t = jnp.tile(scale, (4, 1))

    xpad = jnp.concatenate([jnp.zeros((16, TBL), jnp.bfloat16), xbf], axis=0)
    w2 = w2_ref[...]

    feats = []
    for pi in range(8):
        ra = xpad[32 * pi:32 * pi + 32, :]
        rb = xpad[32 * pi + 16:32 * pi + 48, :]
        for nk in range(2):
            wa = w2[256 * nk:256 * nk + 256, :]
            wb = w2[512 + 256 * nk:768 + 256 * nk, :]
            m = jnp.maximum(
                jnp.dot(wa, ra, preferred_element_type=jnp.float32),
                jnp.dot(wa, rb, preferred_element_type=jnp.float32))
            m2 = jnp.maximum(
                jnp.dot(wb, ra, preferred_element_type=jnp.float32),
                jnp.dot(wb, rb, preferred_element_type=jnp.float32))
            pooled = jnp.maximum(m, m2)
            feats.append(
                jnp.maximum((pooled - mean_t) * scale_t, 0.0)
                .astype(jnp.bfloat16))
    feats_all = jnp.concatenate(feats, axis=0)

    gates = jnp.dot(wg_ref[...], feats_all,
                    preferred_element_type=jnp.float32) + bg_ref[...]
    sig = pl.reciprocal(1.0 + jnp.exp(-gates), approx=True)
    tnh = jnp.tanh(gates)
    cell = sig[0:HIDDEN, :] * tnh[HIDDEN:2 * HIDDEN, :]
    h = sig[2 * HIDDEN:3 * HIDDEN, :] * jnp.tanh(cell)
    o_ref[...] = (jnp.sum(h * wfc_ref[...], axis=0, keepdims=True)
                  + bfc_ref[...])


def _prep_weights(conv_w, w_ih, b_ih, b_hh, w_fc, b_fc):
    w = conv_w[:, 0].astype(jnp.bfloat16).astype(jnp.float32)

    jj = jnp.arange(16)
    jcol = jnp.arange(16)
    w2 = jnp.zeros((2, 16, 16, C), jnp.float32)
    for dj in (0, 1):
        sel = (jj[:, None] == (jcol[None, :] - 1 + dj)).astype(jnp.float32)
        w2 = w2 + sel[None, :, :, None] * w[:, :, dj].T[:, None, None, :]
    w2 = (w2.reshape(2, 16, 8, 2, C).transpose(0, 1, 3, 2, 4)
          .reshape(32, 1024).T).astype(jnp.bfloat16)

    igo = jnp.concatenate([w_ih[0:HIDDEN], w_ih[2 * HIDDEN:4 * HIDDEN]],
                          axis=0)
    wg = (igo.reshape(NG, C, 8, 8)
          .transpose(0, 2, 3, 1)
          .reshape(NG, 4096)).astype(jnp.bfloat16)
    b_all = (b_ih + b_hh).astype(jnp.float32)
    bg = jnp.concatenate([b_all[0:HIDDEN], b_all[2 * HIDDEN:4 * HIDDEN]]
                         ).reshape(NG, 1)

    w00, w01 = w[:, 0, 0], w[:, 0, 1]
    w10, w11 = w[:, 1, 0], w[:, 1, 1]
    sc = jnp.stack([
        w00 + w01 + w10 + w11,
        w00 * w00 + w01 * w01 + w10 * w10 + w11 * w11,
        2.0 * (w00 * w01 + w10 * w11),
        2.0 * (w00 * w10 + w01 * w11),
        2.0 * (w00 * w11),
        2.0 * (w01 * w10),
        jnp.zeros_like(w00),
        jnp.zeros_like(w00),
    ], axis=1) * (1.0 / SP)

    wfc = jnp.broadcast_to(w_fc.reshape(HIDDEN, 1).astype(jnp.float32),
                           (HIDDEN, TBL))
    bfc = jnp.broadcast_to(b_fc.reshape(1, 1).astype(jnp.float32), (1, TBL))
    return w2, wg, bg, sc, wfc, bfc


@functools.partial(jax.jit, static_argnames=())
def kernel(x, conv_w, conv_b, w_ih, b_ih, w_hh, b_hh, w_fc, b_fc):
    del conv_b, w_hh
    B = x.shape[0]
    nt = -(-B // TBL)
    b_pad = nt * TBL
    xt = x[:, 0].transpose(1, 2, 0).reshape(H * W, B)
    if b_pad != B:
        xt = jnp.pad(xt, ((0, 0), (0, b_pad - B)))

    w2, wg, bg, sc, wfc, bfc = _prep_weights(conv_w, w_ih, b_ih, b_hh,
                                             w_fc, b_fc)

    out = pl.pallas_call(
        _fused_kernel,
        out_shape=jax.ShapeDtypeStruct((1, b_pad), jnp.float32),
        grid=(nt,),
        in_specs=[
            pl.BlockSpec((H * W, TBL), lambda i: (0, i)),
            pl.BlockSpec((1024, 32), lambda i: (0, 0)),
            pl.BlockSpec((NG, 4096), lambda i: (0, 0)),
            pl.BlockSpec((NG, 1), lambda i: (0, 0)),
            pl.BlockSpec((C, 8), lambda i: (0, 0)),
            pl.BlockSpec((HIDDEN, TBL), lambda i: (0, 0)),
            pl.BlockSpec((1, TBL), lambda i: (0, 0)),
        ],
        out_specs=pl.BlockSpec((1, TBL), lambda i: (0, i)),
        compiler_params=pltpu.CompilerParams(
            dimension_semantics=("parallel",),
            vmem_limit_bytes=100 << 20),
    )(xt, w2, wg, bg, sc, wfc, bfc)
    return out.reshape(b_pad, 1)[:B]


# speedup vs baseline: 1.0703x; 1.0703x over previous
import functools

import jax
import jax.numpy as jnp
from jax.experimental import pallas as pl
from jax.experimental.pallas import tpu as pltpu

H = W = 16
C = 64
SP = 289
HIDDEN = 32
NG = 3 * HIDDEN
EPS = 1e-5
PL = 256
NPANEL = 2
TBL = PL * NPANEL


def _panel(xs, w2, wg_ref, bg_ref, sc_ref, wfc_ref, bfc_ref):
    xbf = xs.astype(jnp.bfloat16)
    xb = xbf.astype(jnp.float32)
    row = jax.lax.broadcasted_iota(jnp.int32, (256, PL), 0)
    sub = row % 16
    zero = jnp.zeros_like(xb)

    s10 = jnp.where(sub != 0, jnp.roll(xb, 1, axis=0), zero)
    s01 = jnp.where(row >= 16, jnp.roll(xb, 16, axis=0), zero)
    s00 = jnp.where((row >= 16) & (sub != 0), jnp.roll(xb, 17, axis=0), zero)
    s1m = jnp.where((row >= 16) & (sub != 15), jnp.roll(xb, 15, axis=0), zero)
    ssum = jnp.sum(xb, axis=0, keepdims=True)
    c00 = jnp.sum(xb * xb, axis=0, keepdims=True)
    c01 = jnp.sum(xb * s10, axis=0, keepdims=True)
    c10 = jnp.sum(xb * s01, axis=0, keepdims=True)
    c11 = jnp.sum(xb * s00, axis=0, keepdims=True)
    c1m = jnp.sum(xb * s1m, axis=0, keepdims=True)

    mean = sc_ref[:, 0:1] * ssum
    ex2 = (sc_ref[:, 1:2] * c00 + sc_ref[:, 2:3] * c01
           + sc_ref[:, 3:4] * c10 + sc_ref[:, 4:5] * c11
           + sc_ref[:, 5:6] * c1m)
    scale = jax.lax.rsqrt(ex2 - mean * mean + EPS)
    mean_t = jnp.tile(mean, (4, 1))
    scale_t = jnp.tile(scale, (4, 1))

    xpad = jnp.concatenate([jnp.zeros((16, PL), jnp.bfloat16), xbf], axis=0)

    feats = []
    for pi in range(8):
        ra = xpad[32 * pi:32 * pi + 32, :]
        rb = xpad[32 * pi + 16:32 * pi + 48, :]
        for nk in range(2):
            wa = w2[256 * nk:256 * nk + 256, :]
            wb = w2[512 + 256 * nk:768 + 256 * nk, :]
            m = jnp.maximum(
                jnp.dot(wa, ra, preferred_element_type=jnp.float32),
                jnp.dot(wa, rb, preferred_element_type=jnp.float32))
            m2 = jnp.maximum(
                jnp.dot(wb, ra, preferred_element_type=jnp.float32),
                jnp.dot(wb, rb, preferred_element_type=jnp.float32))
            pooled = jnp.maximum(m, m2)
            feats.append(
                jnp.maximum((pooled - mean_t) * scale_t, 0.0)
                .astype(jnp.bfloat16))
    feats_all = jnp.concatenate(feats, axis=0)

    gates = jnp.dot(wg_ref[...], feats_all,
                    preferred_element_type=jnp.float32) + bg_ref[...]
    sig = pl.reciprocal(1.0 + jnp.exp(-gates), approx=True)
    tnh = jnp.tanh(gates)
    cell = sig[0:HIDDEN, :] * tnh[HIDDEN:2 * HIDDEN, :]
    h = sig[2 * HIDDEN:3 * HIDDEN, :] * jnp.tanh(cell)
    return (jnp.sum(h * wfc_ref[...], axis=0, keepdims=True)
            + bfc_ref[...])


def _fused_kernel(x_ref, w2_ref, wg_ref, bg_ref, sc_ref, wfc_ref, bfc_ref,
                  o_ref):
    w2 = w2_ref[...]
    o_ref[...] = jnp.concatenate(
        [_panel(x_ref[:, PL * k:PL * k + PL], w2, wg_ref, bg_ref, sc_ref,
                wfc_ref, bfc_ref)
         for k in range(NPANEL)], axis=1)


def _prep_weights(conv_w, w_ih, b_ih, b_hh, w_fc, b_fc):
    w = conv_w[:, 0].astype(jnp.bfloat16).astype(jnp.float32)

    jj = jnp.arange(16)
    jcol = jnp.arange(16)
    w2 = jnp.zeros((2, 16, 16, C), jnp.float32)
    for dj in (0, 1):
        sel = (jj[:, None] == (jcol[None, :] - 1 + dj)).astype(jnp.float32)
        w2 = w2 + sel[None, :, :, None] * w[:, :, dj].T[:, None, None, :]
    w2 = (w2.reshape(2, 16, 8, 2, C).transpose(0, 1, 3, 2, 4)
          .reshape(32, 1024).T).astype(jnp.bfloat16)

    igo = jnp.concatenate([w_ih[0:HIDDEN], w_ih[2 * HIDDEN:4 * HIDDEN]],
                          axis=0)
    wg = (igo.reshape(NG, C, 8, 8)
          .transpose(0, 2, 3, 1)
          .reshape(NG, 4096)).astype(jnp.bfloat16)
    b_all = (b_ih + b_hh).astype(jnp.float32)
    bg = jnp.concatenate([b_all[0:HIDDEN], b_all[2 * HIDDEN:4 * HIDDEN]]
                         ).reshape(NG, 1)

    w00, w01 = w[:, 0, 0], w[:, 0, 1]
    w10, w11 = w[:, 1, 0], w[:, 1, 1]
    sc = jnp.stack([
        w00 + w01 + w10 + w11,
        w00 * w00 + w01 * w01 + w10 * w10 + w11 * w11,
        2.0 * (w00 * w01 + w10 * w11),
        2.0 * (w00 * w10 + w01 * w11),
        2.0 * (w00 * w11),
        2.0 * (w01 * w10),
        jnp.zeros_like(w00),
        jnp.zeros_like(w00),
    ], axis=1) * (1.0 / SP)

    wfc = jnp.broadcast_to(w_fc.reshape(HIDDEN, 1).astype(jnp.float32),
                           (HIDDEN, PL))
    bfc = jnp.broadcast_to(b_fc.reshape(1, 1).astype(jnp.float32), (1, PL))
    return w2, wg, bg, sc, wfc, bfc


@functools.partial(jax.jit, static_argnames=())
def kernel(x, conv_w, conv_b, w_ih, b_ih, w_hh, b_hh, w_fc, b_fc):
    del conv_b, w_hh
    B = x.shape[0]
    nt = -(-B // TBL)
    b_pad = nt * TBL
    xt = x[:, 0].transpose(1, 2, 0).reshape(H * W, B)
    if b_pad != B:
        xt = jnp.pad(xt, ((0, 0), (0, b_pad - B)))

    w2, wg, bg, sc, wfc, bfc = _prep_weights(conv_w, w_ih, b_ih, b_hh,
                                             w_fc, b_fc)

    out = pl.pallas_call(
        _fused_kernel,
        out_shape=jax.ShapeDtypeStruct((1, b_pad), jnp.float32),
        grid=(nt,),
        in_specs=[
            pl.BlockSpec((H * W, TBL), lambda i: (0, i)),
            pl.BlockSpec((1024, 32), lambda i: (0, 0)),
            pl.BlockSpec((NG, 4096), lambda i: (0, 0)),
            pl.BlockSpec((NG, 1), lambda i: (0, 0)),
            pl.BlockSpec((C, 8), lambda i: (0, 0)),
            pl.BlockSpec((HIDDEN, PL), lambda i: (0, 0)),
            pl.BlockSpec((1, PL), lambda i: (0, 0)),
        ],
        out_specs=pl.BlockSpec((1, TBL), lambda i: (0, i)),
        compiler_params=pltpu.CompilerParams(
            dimension_semantics=("parallel",),
            vmem_limit_bytes=100 << 20),
    )(xt, w2, wg, bg, sc, wfc, bfc)
    return out.reshape(b_pad, 1)[:B]


# speedup vs baseline: 1.1143x; 1.0411x over previous
import functools

import jax
import jax.numpy as jnp
from jax.experimental import pallas as pl
from jax.experimental.pallas import tpu as pltpu

H = W = 16
C = 64
SP = 289
HIDDEN = 32
NG = 3 * HIDDEN
EPS = 1e-5
PL = 256
NPANEL = 4
TBL = PL * NPANEL


def _panel(xs, w2, wg_ref, bg_ref, sc_ref, wfc_ref, bfc_ref):
    xbf = xs.astype(jnp.bfloat16)
    xb = xbf.astype(jnp.float32)
    row = jax.lax.broadcasted_iota(jnp.int32, (256, PL), 0)
    sub = row % 16
    zero = jnp.zeros_like(xb)

    s10 = jnp.where(sub != 0, jnp.roll(xb, 1, axis=0), zero)
    s01 = jnp.where(row >= 16, jnp.roll(xb, 16, axis=0), zero)
    s00 = jnp.where((row >= 16) & (sub != 0), jnp.roll(xb, 17, axis=0), zero)
    s1m = jnp.where((row >= 16) & (sub != 15), jnp.roll(xb, 15, axis=0), zero)
    ssum = jnp.sum(xb, axis=0, keepdims=True)
    c00 = jnp.sum(xb * xb, axis=0, keepdims=True)
    c01 = jnp.sum(xb * s10, axis=0, keepdims=True)
    c10 = jnp.sum(xb * s01, axis=0, keepdims=True)
    c11 = jnp.sum(xb * s00, axis=0, keepdims=True)
    c1m = jnp.sum(xb * s1m, axis=0, keepdims=True)

    mean = sc_ref[:, 0:1] * ssum
    ex2 = (sc_ref[:, 1:2] * c00 + sc_ref[:, 2:3] * c01
           + sc_ref[:, 3:4] * c10 + sc_ref[:, 4:5] * c11
           + sc_ref[:, 5:6] * c1m)
    scale = jax.lax.rsqrt(ex2 - mean * mean + EPS)
    mean_t = jnp.tile(mean, (4, 1))
    scale_t = jnp.tile(scale, (4, 1))

    xpad = jnp.concatenate([jnp.zeros((16, PL), jnp.bfloat16), xbf], axis=0)

    feats = []
    for pi in range(8):
        ra = xpad[32 * pi:32 * pi + 32, :]
        rb = xpad[32 * pi + 16:32 * pi + 48, :]
        for nk in range(2):
            wa = w2[256 * nk:256 * nk + 256, :]
            wb = w2[512 + 256 * nk:768 + 256 * nk, :]
            m = jnp.maximum(
                jnp.dot(wa, ra, preferred_element_type=jnp.float32),
                jnp.dot(wa, rb, preferred_element_type=jnp.float32))
            m2 = jnp.maximum(
                jnp.dot(wb, ra, preferred_element_type=jnp.float32),
                jnp.dot(wb, rb, preferred_element_type=jnp.float32))
            pooled = jnp.maximum(m, m2)
            feats.append(
                jnp.maximum((pooled - mean_t) * scale_t, 0.0)
                .astype(jnp.bfloat16))
    feats_all = jnp.concatenate(feats, axis=0)

    gates = jnp.dot(wg_ref[...], feats_all,
                    preferred_element_type=jnp.float32) + bg_ref[...]
    sig = pl.reciprocal(1.0 + jnp.exp(-gates), approx=True)
    tnh = jnp.tanh(gates)
    cell = sig[0:HIDDEN, :] * tnh[HIDDEN:2 * HIDDEN, :]
    h = sig[2 * HIDDEN:3 * HIDDEN, :] * jnp.tanh(cell)
    return (jnp.sum(h * wfc_ref[...], axis=0, keepdims=True)
            + bfc_ref[...])


def _fused_kernel(x_ref, w2_ref, wg_ref, bg_ref, sc_ref, wfc_ref, bfc_ref,
                  o_ref):
    w2 = w2_ref[...]
    o_ref[...] = jnp.concatenate(
        [_panel(x_ref[:, PL * k:PL * k + PL], w2, wg_ref, bg_ref, sc_ref,
                wfc_ref, bfc_ref)
         for k in range(NPANEL)], axis=1)


def _prep_weights(conv_w, w_ih, b_ih, b_hh, w_fc, b_fc):
    w = conv_w[:, 0].astype(jnp.bfloat16).astype(jnp.float32)

    jj = jnp.arange(16)
    jcol = jnp.arange(16)
    w2 = jnp.zeros((2, 16, 16, C), jnp.float32)
    for dj in (0, 1):
        sel = (jj[:, None] == (jcol[None, :] - 1 + dj)).astype(jnp.float32)
        w2 = w2 + sel[None, :, :, None] * w[:, :, dj].T[:, None, None, :]
    w2 = (w2.reshape(2, 16, 8, 2, C).transpose(0, 1, 3, 2, 4)
          .reshape(32, 1024).T).astype(jnp.bfloat16)

    igo = jnp.concatenate([w_ih[0:HIDDEN], w_ih[2 * HIDDEN:4 * HIDDEN]],
                          axis=0)
    wg = (igo.reshape(NG, C, 8, 8)
          .transpose(0, 2, 3, 1)
          .reshape(NG, 4096)).astype(jnp.bfloat16)
    b_all = (b_ih + b_hh).astype(jnp.float32)
    bg = jnp.concatenate([b_all[0:HIDDEN], b_all[2 * HIDDEN:4 * HIDDEN]]
                         ).reshape(NG, 1)

    w00, w01 = w[:, 0, 0], w[:, 0, 1]
    w10, w11 = w[:, 1, 0], w[:, 1, 1]
    sc = jnp.stack([
        w00 + w01 + w10 + w11,
        w00 * w00 + w01 * w01 + w10 * w10 + w11 * w11,
        2.0 * (w00 * w01 + w10 * w11),
        2.0 * (w00 * w10 + w01 * w11),
        2.0 * (w00 * w11),
        2.0 * (w01 * w10),
        jnp.zeros_like(w00),
        jnp.zeros_like(w00),
    ], axis=1) * (1.0 / SP)

    wfc = jnp.broadcast_to(w_fc.reshape(HIDDEN, 1).astype(jnp.float32),
                           (HIDDEN, PL))
    bfc = jnp.broadcast_to(b_fc.reshape(1, 1).astype(jnp.float32), (1, PL))
    return w2, wg, bg, sc, wfc, bfc


@functools.partial(jax.jit, static_argnames=())
def kernel(x, conv_w, conv_b, w_ih, b_ih, w_hh, b_hh, w_fc, b_fc):
    del conv_b, w_hh
    B = x.shape[0]
    nt = -(-B // TBL)
    b_pad = nt * TBL
    xt = x[:, 0].transpose(1, 2, 0).reshape(H * W, B)
    if b_pad != B:
        xt = jnp.pad(xt, ((0, 0), (0, b_pad - B)))

    w2, wg, bg, sc, wfc, bfc = _prep_weights(conv_w, w_ih, b_ih, b_hh,
                                             w_fc, b_fc)

    out = pl.pallas_call(
        _fused_kernel,
        out_shape=jax.ShapeDtypeStruct((1, b_pad), jnp.float32),
        grid=(nt,),
        in_specs=[
            pl.BlockSpec((H * W, TBL), lambda i: (0, i)),
            pl.BlockSpec((1024, 32), lambda i: (0, 0)),
            pl.BlockSpec((NG, 4096), lambda i: (0, 0)),
            pl.BlockSpec((NG, 1), lambda i: (0, 0)),
            pl.BlockSpec((C, 8), lambda i: (0, 0)),
            pl.BlockSpec((HIDDEN, PL), lambda i: (0, 0)),
            pl.BlockSpec((1, PL), lambda i: (0, 0)),
        ],
        out_specs=pl.BlockSpec((1, TBL), lambda i: (0, i)),
        compiler_params=pltpu.CompilerParams(
            dimension_semantics=("parallel",),
            vmem_limit_bytes=100 << 20),
    )(xt, w2, wg, bg, sc, wfc, bfc)
    return out.reshape(b_pad, 1)[:B]
